# baseline (speedup 1.0000x reference)
import functools

import jax
import jax.numpy as jnp
from jax import lax
from jax.experimental import pallas as pl
from jax.experimental.pallas import tpu as pltpu
from jax.experimental.pallas import tpu_sc as plsc


def _sc_gather(x_flat, idx_flat_raw, n_cols):
    total = idx_flat_raw.shape[0]
    info = plsc.get_sparse_core_info()
    num_workers = info.num_cores * info.num_subcores
    chunk = total // num_workers
    lanes = info.num_lanes

    mesh = plsc.VectorSubcoreMesh(core_axis_name="c", subcore_axis_name="s")

    @functools.partial(
        pl.kernel,
        mesh=mesh,
        out_type=jax.ShapeDtypeStruct((total,), jnp.float32),
        scratch_types=[
            pltpu.VMEM((chunk,), jnp.int32),
            pltpu.VMEM((chunk,), jnp.float32),
            pltpu.SemaphoreType.DMA,
        ],
    )
    def gather_kernel(x_hbm, idx_hbm, out_hbm, idx_v, out_v, sem):
        wid = lax.axis_index("s") * info.num_cores + lax.axis_index("c")
        base = wid * chunk
        pltpu.sync_copy(idx_hbm.at[pl.ds(base, chunk)], idx_v)

        col_iota = lax.iota(jnp.int32, lanes)
        lanes_c = jnp.int32(lanes)
        ncols_c = jnp.int32(n_cols)

        def body(_, o):
            v = idx_v[pl.ds(o, lanes)]
            col = col_iota + lax.rem(o, ncols_c)
            idx_v[pl.ds(o, lanes)] = v * ncols_c + col
            return o + lanes_c

        lax.fori_loop(0, chunk // lanes, body, jnp.int32(0))

        pltpu.async_copy(x_hbm.at[idx_v], out_v, sem).wait()
        pltpu.sync_copy(out_v, out_hbm.at[pl.ds(base, chunk)])

    return gather_kernel(x_flat, idx_flat_raw)


def kernel(x, dim, index, sparse_grad):
    del dim, sparse_grad
    n_rows, n_cols = x.shape
    b, c = index.shape
    out = _sc_gather(
        x.reshape(-1),
        index.astype(jnp.int32).reshape(-1),
        n_cols,
    )
    return out.reshape(b, c)


# speedup vs baseline: 2.8412x; 2.8412x over previous
import functools

import jax
import jax.numpy as jnp
from jax import lax
from jax.experimental import pallas as pl
from jax.experimental.pallas import tpu as pltpu
from jax.experimental.pallas import tpu_sc as plsc

_LW_BITS = 17
_LW = 1 << _LW_BITS


def _detile_body(x_ref, o_ref):
    o_ref[...] = x_ref[...].reshape(8 * _LW)


def _detile_strip(xt, g, n_windows):
    return pl.pallas_call(
        _detile_body,
        grid=(n_windows,),
        in_specs=[pl.BlockSpec((8, _LW), lambda w, g=g: (jnp.int32(g), w))],
        out_specs=pl.BlockSpec((8 * _LW,), lambda w: (w,)),
        out_shape=jax.ShapeDtypeStruct((n_windows * 8 * _LW,), jnp.float32),
    )(xt)


def _sc_gather_strip(scratch_g, idx1d, g, b):
    info = plsc.get_sparse_core_info()
    num_workers = info.num_cores * info.num_subcores
    lanes = info.num_lanes
    per_tile = 8 * b // num_workers
    quarters = num_workers // 8

    mesh = plsc.VectorSubcoreMesh(core_axis_name="c", subcore_axis_name="s")

    @functools.partial(
        pl.kernel,
        mesh=mesh,
        out_type=jax.ShapeDtypeStruct((8 * b,), jnp.float32),
        scratch_types=[
            pltpu.VMEM((per_tile,), jnp.int32),
            pltpu.VMEM((per_tile,), jnp.float32),
            pltpu.SemaphoreType.DMA,
        ],
    )
    def gather_kernel(scratch_hbm, idx_hbm, out_hbm, idx_v, val_v, sem):
        wid = lax.axis_index("s") * info.num_cores + lax.axis_index("c")
        r = lax.div(wid, jnp.int32(quarters))
        q = lax.rem(wid, jnp.int32(quarters))
        base = (jnp.int32(8 * g) + r) * jnp.int32(b) + q * jnp.int32(per_tile)
        out_base = r * jnp.int32(b) + q * jnp.int32(per_tile)
        pltpu.sync_copy(idx_hbm.at[pl.ds(base, per_tile)], idx_v)

        rshift = r * jnp.int32(1 << _LW_BITS)
        mask = jnp.int32(_LW - 1)

        def body(_, o):
            v = idx_v[pl.ds(o, lanes)]
            w = lax.shift_right_logical(v, jnp.int32(_LW_BITS))
            off = (lax.shift_left(w, jnp.int32(20))
                   + (v & mask) + rshift)
            idx_v[pl.ds(o, lanes)] = off
            return o + jnp.int32(lanes)

        lax.fori_loop(0, per_tile // lanes, body, jnp.int32(0))

        pltpu.async_copy(scratch_hbm.at[idx_v], val_v, sem).wait()
        pltpu.sync_copy(val_v, out_hbm.at[pl.ds(out_base, per_tile)])

    return gather_kernel(scratch_g, idx1d)


def kernel(x, dim, index, sparse_grad):
    del dim, sparse_grad
    n_rows, n_cols = x.shape
    b, c = index.shape
    xt = x.T
    idx1d = index.T.astype(jnp.int32).reshape(-1)
    n_strips = n_cols // 8
    n_windows = -(-n_rows // _LW)

    outs = []
    for g in range(n_strips):
        scratch_g = _detile_strip(xt, g, n_windows)
        outs.append(_sc_gather_strip(scratch_g, idx1d, g, b))
    out_t = jnp.concatenate(outs).reshape(c, b)
    return out_t.T
